# Initial kernel scaffold; baseline (speedup 1.0000x reference)
import jax
import jax.numpy as jnp
from jax.experimental import pallas as pl


def kernel(states, relations, alive_mask, action_mask, action, se_W1, se_b1, se_W2, se_b2, re_W1, re_b1, re_W2, re_b2, ra_W, ra_b, ae_W1, ae_b1, ae_W2, ae_b2, action_embed, de_W1, de_b1, de_W2, de_b2, le_W1, le_b1, le_W2, le_b2):
    raise NotImplementedError("write your pallas kernel here")


# speedup vs baseline: 4.0061x; 4.0061x over previous
import functools

import jax
import jax.numpy as jnp
from jax.experimental import pallas as pl

B, N, S, R, H, A = 64, 12, 128, 128, 256, 19
BALL = N - 1


def _relu(x):
    return jnp.maximum(x, 0.0)


def _dot(a, b):
    return jax.lax.dot_general(a, b, (((1,), (0,)), ((), ())),
                               preferred_element_type=jnp.float32)


def _grface_kernel(states_ref, relations_ref, alive_ref, action_ref,
                   se_W1_ref, se_b1_ref, se_W2_ref, se_b2_ref,
                   re_W1_ref, re_b1_ref, re_W2_ref, re_b2_ref,
                   ra_W_ref, ra_b_ref,
                   ae_W1_ref, ae_b1_ref, ae_W2_ref, ae_b2_ref,
                   aemb_ref,
                   de_W1_ref, de_b1_ref, de_W2_ref, de_b2_ref,
                   le_W1_ref, le_b1_ref, le_W2_ref, le_b2_ref,
                   out_ref, *, bt):
    P = bt * N
    st = states_ref[...]
    own = st[:, :, 4]
    am = alive_ref[...]
    act = action_ref[...]

    se_W1 = se_W1_ref[...]; se_b1 = se_b1_ref[...]
    se_W2 = se_W2_ref[...]; se_b2 = se_b2_ref[...]
    re_W1 = re_W1_ref[...]; re_b1 = re_b1_ref[...]
    re_W2 = re_W2_ref[...]; re_b2 = re_b2_ref[...]
    ra_W = ra_W_ref[...]; ra_b = ra_b_ref[...]
    ae_W1 = ae_W1_ref[...]; ae_b1 = ae_b1_ref[...]
    ae_W2 = ae_W2_ref[...]; ae_b2 = ae_b2_ref[...]
    aemb = aemb_ref[...]
    de_W1 = de_W1_ref[...]; de_b1 = de_b1_ref[...]
    de_W2 = de_W2_ref[...]; de_b2 = de_b2_ref[...]
    le_W1 = le_W1_ref[...]; le_b1 = le_b1_ref[...]
    le_W2 = le_W2_ref[...]; le_b2 = le_b2_ref[...]

    x = st.reshape(P, S)
    h = _relu(_dot(x, se_W1) + se_b1)
    state_se = _relu(_dot(h, se_W2) + se_b2)

    rel = relations_ref[...]
    r1a = _dot(rel.reshape(bt * N * N, R), re_W1[:R])
    r1b = _dot(state_se, re_W1[R:])
    rel1 = _relu(r1a.reshape(bt, N, N, H)
                 + r1b.reshape(bt, 1, N, H) + re_b1)
    rel2 = _relu(_dot(rel1.reshape(bt * N * N, H), re_W2) + re_b2)
    rel2 = rel2.reshape(bt, N, N, 2 * H) * am[:, None, :, None]
    rel_avr = jnp.sum(rel2[..., :H], axis=2) * (1.0 / N)
    rel_max = jnp.max(rel2[..., H:], axis=2)

    s0 = (_dot(state_se, ra_W[:H])
          + _dot(rel_avr.reshape(P, H), ra_W[H:2 * H])
          + _dot(rel_max.reshape(P, H), ra_W[2 * H:]) + ra_b)
    state0 = _relu(s0).reshape(bt, N, H)

    s0f = state0.reshape(P, H)
    h_ae = _relu(_dot(s0f, ae_W1) + ae_b1)
    ae_out = _relu(_dot(h_ae, ae_W2) + ae_b2)
    pe0 = ae_out.reshape(bt, N, A, H) * own[:, :, None, None]

    onehot = (act[:, :, None]
              == jax.lax.broadcasted_iota(jnp.int32, (bt, N, A), 2)
              ).astype(jnp.float32)
    pe_g = jnp.sum(pe0 * onehot[..., None], axis=2)

    sball_rows = [state0[:, BALL]]
    for j in range(N - 1):
        sball_rows.append(sball_rows[-1] + pe_g[:, j])
    sball = jnp.stack(sball_rows, axis=1)
    sball11 = sball_rows[BALL]

    h2 = _relu(_dot(sball11, ae_W1) + ae_b1)
    aeo2 = _relu(_dot(h2, ae_W2) + ae_b2)
    pe_ball = aeo2.reshape(bt, 1, A, H) * own[:, BALL][:, None, None, None]
    pe = jnp.concatenate([pe0[:, :BALL], pe_ball], axis=1)

    xact = jnp.concatenate([state0[:, :BALL], sball11[:, None]], axis=1)

    ae_emb_g = _dot(onehot.reshape(P, A), aemb)
    state_post = s0f + ae_emb_g
    dcat = jnp.concatenate([s0f, state_post], axis=0)
    hD = _relu(_dot(dcat, de_W1) + de_b1)
    D = _relu(_dot(hD, de_W2) + de_b2)
    Dm = D.reshape(2, bt, N, 2 * H) * am[None, :, :, None]
    Dpre, Dpost = Dm[0], Dm[1]

    neg = jnp.full((bt, H), -jnp.inf, jnp.float32)
    zero = jnp.zeros((bt, H), jnp.float32)
    post_cum = [zero]
    post_mx = [neg]
    for j in range(N - 1):
        post_cum.append(post_cum[-1] + Dpost[:, j, :H])
        post_mx.append(jnp.maximum(post_mx[-1], Dpost[:, j, H:]))
    pre_suf = [zero] * (N + 1)
    pre_mx = [neg] * (N + 1)
    for i in range(N - 2, -1, -1):
        pre_suf[i] = pre_suf[i + 1] + (Dpre[:, i + 1, :H]
                                       if i + 1 <= N - 2 else zero)
        pre_mx[i] = (jnp.maximum(pre_mx[i + 1], Dpre[:, i + 1, H:])
                     if i + 1 <= N - 2 else pre_mx[i + 1])
    others_avr = jnp.stack([post_cum[i] + pre_suf[i] for i in range(N)], axis=1)
    others_max = jnp.stack([jnp.maximum(post_mx[i], pre_mx[i])
                            for i in range(N)], axis=1)

    E1 = _dot(aemb, de_W1)
    xact_l1 = _dot(xact.reshape(P, H), de_W1) + de_b1
    A1 = _relu(xact_l1.reshape(bt, N, 1, H) + E1.reshape(1, 1, A, H))
    act_dec = _relu(_dot(A1.reshape(P * A, H), de_W2) + de_b2)
    act_dec = act_dec.reshape(bt, N, A, 2 * H)

    pe_l1 = _dot(pe.reshape(P * A, H), de_W1)
    sball_l1 = _dot(sball.reshape(P, H), de_W1) + de_b1
    P1 = _relu(sball_l1.reshape(bt, N, 1, H) + pe_l1.reshape(bt, N, A, H))
    pas_dec = _relu(_dot(P1.reshape(P * A, H), de_W2) + de_b2)
    pas_dec = pas_dec.reshape(bt, N, A, 2 * H)

    am_i = am[:, :, None, None]
    am_ball = am[:, BALL][:, None, None, None]
    use_act = (jax.lax.broadcasted_iota(jnp.int32, (1, N, 1, 1), 1)
               < BALL)
    act_avr = jnp.where(use_act, am_i * act_dec[..., :H], 0.0)
    act_max = jnp.where(use_act, am_i * act_dec[..., H:], -jnp.inf)
    dec_avr = (others_avr[:, :, None, :] + act_avr
               + am_ball * pas_dec[..., :H]) * (1.0 / N)
    dec_max = jnp.maximum(jnp.maximum(others_max[:, :, None, :], act_max),
                          am_ball * pas_dec[..., H:])

    d2a = dec_avr.reshape(P * A, H)
    d2b = dec_max.reshape(P * A, H)
    l1 = _relu(_dot(d2a, le_W1[:H]) + _dot(d2b, le_W1[H:]) + le_b1)
    logit = _dot(l1, le_W2) + le_b2
    out_ref[...] = logit.reshape(bt, N, A)


@functools.partial(jax.jit, static_argnames=("interpret",))
def _run(states, relations, alive_mask, action,
         se_W1, se_b1, se_W2, se_b2, re_W1, re_b1, re_W2, re_b2,
         ra_W, ra_b, ae_W1, ae_b1, ae_W2, ae_b2, aemb,
         de_W1, de_b1, de_W2, de_b2, le_W1, le_b1, le_W2, le_b2,
         interpret=False):
    bt = 8
    grid = (B // bt,)

    def bmap(i):
        return (i, 0, 0)

    weights = (se_W1, se_b1.reshape(1, H), se_W2, se_b2.reshape(1, H),
               re_W1, re_b1.reshape(1, H), re_W2, re_b2.reshape(1, 2 * H),
               ra_W, ra_b.reshape(1, H),
               ae_W1, ae_b1.reshape(1, H), ae_W2, ae_b2.reshape(1, A * H),
               aemb,
               de_W1, de_b1.reshape(1, H), de_W2, de_b2.reshape(1, 2 * H),
               le_W1, le_b1.reshape(1, H), le_W2, le_b2.reshape(1, 1))
    w_specs = [pl.BlockSpec(w.shape, lambda i, nd=w.ndim: (0,) * nd)
               for w in weights]
    return pl.pallas_call(
        functools.partial(_grface_kernel, bt=bt),
        grid=grid,
        in_specs=[
            pl.BlockSpec((bt, N, S), bmap),
            pl.BlockSpec((bt, N, N, R), lambda i: (i, 0, 0, 0)),
            pl.BlockSpec((bt, N), lambda i: (i, 0)),
            pl.BlockSpec((bt, N), lambda i: (i, 0)),
        ] + w_specs,
        out_specs=pl.BlockSpec((bt, N, A), bmap),
        out_shape=jax.ShapeDtypeStruct((B, N, A), jnp.float32),
        interpret=interpret,
    )(states, relations, alive_mask, action, *weights)


def kernel(states, relations, alive_mask, action_mask, action,
           se_W1, se_b1, se_W2, se_b2, re_W1, re_b1, re_W2, re_b2,
           ra_W, ra_b, ae_W1, ae_b1, ae_W2, ae_b2, action_embed,
           de_W1, de_b1, de_W2, de_b2, le_W1, le_b1, le_W2, le_b2):
    del action_mask
    aemb = action_embed.reshape(A, H)
    logits = _run(states, relations, alive_mask, action.astype(jnp.int32),
                  se_W1, se_b1, se_W2, se_b2, re_W1, re_b1, re_W2, re_b2,
                  ra_W, ra_b, ae_W1, ae_b1, ae_W2, ae_b2, aemb,
                  de_W1, de_b1, de_W2, de_b2, le_W1, le_b1, le_W2, le_b2)
    return (logits, action)
